# Initial kernel scaffold; baseline (speedup 1.0000x reference)
import jax
import jax.numpy as jnp
from jax.experimental import pallas as pl


def kernel(z, means, devs, mix_partition):
    raise NotImplementedError("write your pallas kernel here")


# speedup vs baseline: 34.4373x; 34.4373x over previous
import dataclasses
import functools

import jax
import jax.numpy as jnp
from jax import lax
from jax.experimental import pallas as pl
from jax.experimental.pallas import tpu as pltpu
from jax.experimental.pallas import tpu_sc as plsc

N = 1000000
D = 8
K = 1024
LANES = 16

CHUNK_ROWS = 800
GROUPS = CHUNK_ROWS // LANES
NUM_CHUNKS = N // CHUNK_ROWS
ZW = D + 1


def _sc_body(z_hbm, means_hbm, devs_hbm, part_hbm, out_hbm,
             part_v, means_v, devs_v, sem):
    c1 = pltpu.async_copy(part_hbm, part_v, sem)
    c2 = pltpu.async_copy(means_hbm, means_v, sem)
    c3 = pltpu.async_copy(devs_hbm, devs_v, sem)
    c1.wait()
    c2.wait()
    c3.wait()

    iota = lax.iota(jnp.int32, LANES)
    iota_z = iota * ZW
    iota_y = iota * D

    def chunk_body(z_v, out_v):
        @pl.loop(0, GROUPS)
        def _(g):
            rowz = g * (LANES * ZW) + iota_z
            u = plsc.load_gather(z_v, [rowz])
            pos = jnp.zeros((LANES,), jnp.int32)
            bit = K // 2
            while bit:
                vals = plsc.load_gather(part_v, [pos + (bit - 1)])
                pos = jnp.where(vals <= u, pos + bit, pos)
                bit //= 2
            idx = jnp.minimum(pos, K - 1)
            xs = [plsc.load_gather(z_v, [rowz + (1 + j)]) for j in range(D)]
            base_m = idx * D
            base_d = idx * (D * D)
            accs = [plsc.load_gather(means_v, [base_m + i]) for i in range(D)]
            for i in range(D):
                acc = accs[i]
                for j in range(D):
                    m = plsc.load_gather(devs_v, [base_d + (i * D + j)])
                    acc = acc + m * xs[j]
                accs[i] = acc
            rowy = g * (LANES * D) + iota_y
            for i in range(D):
                plsc.store_scatter(out_v, [rowy + i], accs[i])

    pltpu.emit_pipeline(
        chunk_body,
        grid=(NUM_CHUNKS,),
        in_specs=[pl.BlockSpec((CHUNK_ROWS * ZW,), lambda i: (i,))],
        out_specs=[pl.BlockSpec((CHUNK_ROWS * D,), lambda i: (i,))],
        core_axis_name=("c", "s"),
        dimension_semantics=(pltpu.PARALLEL,),
    )(z_hbm, out_hbm)


@jax.jit
def kernel(z, means, devs, mix_partition):
    mesh = plsc.VectorSubcoreMesh(core_axis_name="c", subcore_axis_name="s")
    cp = pltpu.CompilerParams()
    if "needs_layout_passes" in pltpu.CompilerParams.__dataclass_fields__:
        cp = dataclasses.replace(cp, needs_layout_passes=False)
    run = pl.kernel(
        _sc_body,
        out_type=jax.ShapeDtypeStruct((N * D,), jnp.float32),
        mesh=mesh,
        scratch_types=[
            pltpu.VMEM((K,), jnp.float32),
            pltpu.VMEM((K * D,), jnp.float32),
            pltpu.VMEM((K * D * D,), jnp.float32),
            pltpu.SemaphoreType.DMA,
        ],
        compiler_params=cp,
    )
    out = run(z.reshape(N * ZW),
              means.reshape(K * D),
              devs.reshape(K * D * D),
              mix_partition)
    return out.reshape(N, D)


# speedup vs baseline: 34.4640x; 1.0008x over previous
import dataclasses
import functools

import jax
import jax.numpy as jnp
from jax import lax
from jax.experimental import pallas as pl
from jax.experimental.pallas import tpu as pltpu
from jax.experimental.pallas import tpu_sc as plsc

N = 1000000
D = 8
K = 1024
LANES = 16

CHUNK_ROWS = 800
GROUPS = CHUNK_ROWS // LANES
NUM_CHUNKS = N // CHUNK_ROWS
ZW = D + 1


def _sc_body(z_hbm, means_hbm, devs_hbm, part_hbm, out_hbm,
             part_v, means_v, devs_v, sem):
    c1 = pltpu.async_copy(part_hbm, part_v, sem)
    c2 = pltpu.async_copy(means_hbm, means_v, sem)
    c3 = pltpu.async_copy(devs_hbm, devs_v, sem)
    c1.wait()
    c2.wait()
    c3.wait()

    iota = lax.iota(jnp.int32, LANES)
    iota_z = iota * ZW
    iota_y = iota * D

    def chunk_body(z_v, out_v):
        @pl.loop(0, GROUPS, unroll=4)
        def _(g):
            rowz = g * (LANES * ZW) + iota_z
            u = plsc.load_gather(z_v, [rowz])
            pos = jnp.zeros((LANES,), jnp.int32)
            bit = K // 2
            while bit:
                vals = plsc.load_gather(part_v, [pos + (bit - 1)])
                pos = jnp.where(vals <= u, pos + bit, pos)
                bit //= 2
            idx = jnp.minimum(pos, K - 1)
            xs = [plsc.load_gather(z_v, [rowz + (1 + j)]) for j in range(D)]
            base_m = idx * D
            base_d = idx * (D * D)
            accs = [plsc.load_gather(means_v, [base_m + i]) for i in range(D)]
            for i in range(D):
                acc = accs[i]
                for j in range(D):
                    m = plsc.load_gather(devs_v, [base_d + (i * D + j)])
                    acc = acc + m * xs[j]
                accs[i] = acc
            rowy = g * (LANES * D) + iota_y
            for i in range(D):
                plsc.store_scatter(out_v, [rowy + i], accs[i])

    pltpu.emit_pipeline(
        chunk_body,
        grid=(NUM_CHUNKS,),
        in_specs=[pl.BlockSpec((CHUNK_ROWS * ZW,), lambda i: (i,))],
        out_specs=[pl.BlockSpec((CHUNK_ROWS * D,), lambda i: (i,))],
        core_axis_name=("c", "s"),
        dimension_semantics=(pltpu.PARALLEL,),
    )(z_hbm, out_hbm)


@jax.jit
def kernel(z, means, devs, mix_partition):
    mesh = plsc.VectorSubcoreMesh(core_axis_name="c", subcore_axis_name="s")
    cp = pltpu.CompilerParams()
    if "needs_layout_passes" in pltpu.CompilerParams.__dataclass_fields__:
        cp = dataclasses.replace(cp, needs_layout_passes=False)
    run = pl.kernel(
        _sc_body,
        out_type=jax.ShapeDtypeStruct((N * D,), jnp.float32),
        mesh=mesh,
        scratch_types=[
            pltpu.VMEM((K,), jnp.float32),
            pltpu.VMEM((K * D,), jnp.float32),
            pltpu.VMEM((K * D * D,), jnp.float32),
            pltpu.SemaphoreType.DMA,
        ],
        compiler_params=cp,
    )
    out = run(z.reshape(N * ZW),
              means.reshape(K * D),
              devs.reshape(K * D * D),
              mix_partition)
    return out.reshape(N, D)


# speedup vs baseline: 36.4183x; 1.0567x over previous
import dataclasses

import jax
import jax.numpy as jnp
from jax import lax
from jax.experimental import pallas as pl
from jax.experimental.pallas import tpu as pltpu
from jax.experimental.pallas import tpu_sc as plsc

N = 1000000
D = 8
K = 1024
LANES = 16

CHUNK_ROWS = 800
GROUPS = CHUNK_ROWS // LANES
NUM_CHUNKS = N // CHUNK_ROWS
ZW = D + 1


def _sc_body(z_hbm, means_hbm, devs_hbm, part_hbm, out_hbm,
             part_v, means_v, devs_v, sem):
    c1 = pltpu.async_copy(part_hbm, part_v, sem)
    c2 = pltpu.async_copy(means_hbm, means_v, sem)
    c3 = pltpu.async_copy(devs_hbm, devs_v, sem)
    c1.wait()
    c2.wait()
    c3.wait()

    iota = lax.iota(jnp.int32, LANES)
    iota_z = iota * ZW
    iota_y = iota * D

    def chunk_body(z_v, out_v):
        @plsc.parallel_loop(0, GROUPS, unroll=4)
        def _(g):
            rowz = g * (LANES * ZW) + iota_z
            u = plsc.load_gather(z_v, [rowz])
            pos = jnp.zeros((LANES,), jnp.int32)
            bit = K // 2
            while bit:
                vals = plsc.load_gather(part_v, [pos + (bit - 1)])
                pos = jnp.where(vals <= u, pos + bit, pos)
                bit //= 2
            idx = jnp.minimum(pos, K - 1)
            xs = [plsc.load_gather(z_v, [rowz + (1 + j)]) for j in range(D)]
            base_m = idx * D
            base_d = idx * (D * D)
            accs = [plsc.load_gather(means_v, [base_m + i]) for i in range(D)]
            for i in range(D):
                acc = accs[i]
                for j in range(D):
                    m = plsc.load_gather(devs_v, [base_d + (i * D + j)])
                    acc = acc + m * xs[j]
                accs[i] = acc
            rowy = g * (LANES * D) + iota_y
            for i in range(D):
                plsc.store_scatter(out_v, [rowy + i], accs[i])

    pltpu.emit_pipeline(
        chunk_body,
        grid=(NUM_CHUNKS,),
        in_specs=[pl.BlockSpec((CHUNK_ROWS * ZW,), lambda i: (i,))],
        out_specs=[pl.BlockSpec((CHUNK_ROWS * D,), lambda i: (i,))],
        core_axis_name=("c", "s"),
        dimension_semantics=(pltpu.PARALLEL,),
    )(z_hbm, out_hbm)


@jax.jit
def kernel(z, means, devs, mix_partition):
    mesh = plsc.VectorSubcoreMesh(core_axis_name="c", subcore_axis_name="s")
    cp = pltpu.CompilerParams()
    if "needs_layout_passes" in pltpu.CompilerParams.__dataclass_fields__:
        cp = dataclasses.replace(cp, needs_layout_passes=False)
    run = pl.kernel(
        _sc_body,
        out_type=jax.ShapeDtypeStruct((N * D,), jnp.float32),
        mesh=mesh,
        scratch_types=[
            pltpu.VMEM((K,), jnp.float32),
            pltpu.VMEM((K * D,), jnp.float32),
            pltpu.VMEM((K * D * D,), jnp.float32),
            pltpu.SemaphoreType.DMA,
        ],
        compiler_params=cp,
    )
    out = run(z.reshape(N * ZW),
              means.reshape(K * D),
              devs.reshape(K * D * D),
              mix_partition)
    return out.reshape(N, D)


# speedup vs baseline: 57.0727x; 1.5671x over previous
import dataclasses

import jax
import jax.numpy as jnp
from jax import lax
from jax.experimental import pallas as pl
from jax.experimental.pallas import tpu as pltpu
from jax.experimental.pallas import tpu_sc as plsc

N = 1000000
D = 8
K = 1024
LANES = 16

CHUNK_ROWS = 800
GROUPS = CHUNK_ROWS // LANES
NUM_CHUNKS = N // CHUNK_ROWS
ZW = D + 1
MPAD = D + 1
DPAD = D * D + 1


def _sc_body(z_hbm, means_hbm, devs_hbm, part_hbm, out_hbm,
             part_v, means_v, devs_v, sem):
    c1 = pltpu.async_copy(part_hbm, part_v, sem)
    c2 = pltpu.async_copy(means_hbm, means_v, sem)
    c3 = pltpu.async_copy(devs_hbm, devs_v, sem)
    c1.wait()
    c2.wait()
    c3.wait()

    iota = lax.iota(jnp.int32, LANES)
    iota_z = iota * ZW
    iota_y = iota * D

    def chunk_body(z_v, out_v):
        @plsc.parallel_loop(0, GROUPS, unroll=4)
        def _(g):
            rowz = g * (LANES * ZW) + iota_z
            u = plsc.load_gather(z_v, [rowz])
            pos = jnp.zeros((LANES,), jnp.int32)
            bit = K // 2
            while bit:
                vals = plsc.load_gather(part_v, [pos + (bit - 1)])
                pos = jnp.where(vals <= u, pos + bit, pos)
                bit //= 2
            idx = jnp.minimum(pos, K - 1)
            xs = [plsc.load_gather(z_v, [rowz + (1 + j)]) for j in range(D)]
            base_m = idx * MPAD
            base_d = idx * DPAD
            accs = [plsc.load_gather(means_v, [base_m + i]) for i in range(D)]
            for i in range(D):
                acc = accs[i]
                for j in range(D):
                    m = plsc.load_gather(devs_v, [base_d + (i * D + j)])
                    acc = acc + m * xs[j]
                accs[i] = acc
            rowy = g * (LANES * D) + iota_y
            for i in range(D):
                plsc.store_scatter(out_v, [rowy + i], accs[i])

    pltpu.emit_pipeline(
        chunk_body,
        grid=(NUM_CHUNKS,),
        in_specs=[pl.BlockSpec((CHUNK_ROWS * ZW,), lambda i: (i,))],
        out_specs=[pl.BlockSpec((CHUNK_ROWS * D,), lambda i: (i,))],
        core_axis_name=("c", "s"),
        dimension_semantics=(pltpu.PARALLEL,),
    )(z_hbm, out_hbm)


@jax.jit
def kernel(z, means, devs, mix_partition):
    mesh = plsc.VectorSubcoreMesh(core_axis_name="c", subcore_axis_name="s")
    cp = pltpu.CompilerParams()
    if "needs_layout_passes" in pltpu.CompilerParams.__dataclass_fields__:
        cp = dataclasses.replace(cp, needs_layout_passes=False)
    run = pl.kernel(
        _sc_body,
        out_type=jax.ShapeDtypeStruct((N * D,), jnp.float32),
        mesh=mesh,
        scratch_types=[
            pltpu.VMEM((K,), jnp.float32),
            pltpu.VMEM((K * MPAD,), jnp.float32),
            pltpu.VMEM((K * DPAD,), jnp.float32),
            pltpu.SemaphoreType.DMA,
        ],
        compiler_params=cp,
    )
    means_p = jnp.pad(means.reshape(K, D), ((0, 0), (0, MPAD - D)))
    devs_p = jnp.pad(devs.reshape(K, D * D), ((0, 0), (0, DPAD - D * D)))
    out = run(z.reshape(N * ZW),
              means_p.reshape(K * MPAD),
              devs_p.reshape(K * DPAD),
              mix_partition)
    return out.reshape(N, D)
